# baseline (speedup 1.0000x reference)
import functools

import jax
import jax.numpy as jnp
from jax import lax
from jax.experimental import pallas as pl
from jax.experimental.pallas import tpu as pltpu
from jax.experimental.pallas import tpu_sc as plsc

_N = 1000000
_M = 16384
_CHUNK = 128
_L = 16

_cached = None


def _build():
    global _cached
    if _cached is not None:
        return _cached

    info = plsc.get_sparse_core_info()
    NC, NS = info.num_cores, info.num_subcores
    NW = NC * NS
    assert _M % (NW * _CHUNK) == 0
    n_chunks = _M // (NW * _CHUNK)
    ids_w = n_chunks * _CHUNK
    qrows_w = ids_w * 4 // _CHUNK
    trows_w = ids_w * 3 // _CHUNK

    mesh = plsc.VectorSubcoreMesh(core_axis_name="c", subcore_axis_name="s")

    @functools.partial(
        pl.kernel,
        mesh=mesh,
        out_type=(
            jax.ShapeDtypeStruct((_M * 4 // _CHUNK, _CHUNK), jnp.float32),
            jax.ShapeDtypeStruct((_M * 3 // _CHUNK, _CHUNK), jnp.float32),
        ),
        scratch_types=[
            pltpu.VMEM((n_chunks, _CHUNK), jnp.int32),
            pltpu.VMEM((n_chunks, _CHUNK), jnp.int32),
            pltpu.VMEM((qrows_w, _CHUNK), jnp.int32),
            pltpu.VMEM((trows_w, _CHUNK), jnp.int32),
            pltpu.VMEM((qrows_w, _CHUNK), jnp.float32),
            pltpu.VMEM((trows_w, _CHUNK), jnp.float32),
            pltpu.SemaphoreType.DMA,
        ],
        compiler_params=pltpu.CompilerParams(
            use_tc_tiling_on_sc=False, needs_layout_passes=False),
    )
    def cam_gather(qf_hbm, tf_hbm, fmi_hbm, fid_hbm, out_q, out_t,
                   idx_v, rid_v, qidx_v, tidx_v, qg_v, tg_v, sem):
        wid = lax.axis_index("s") * NC + lax.axis_index("c")
        pltpu.sync_copy(fid_hbm.at[pl.ds(wid * n_chunks, n_chunks)], idx_v)
        for c in range(n_chunks):
            pltpu.async_copy(fmi_hbm.at[idx_v.at[c]], rid_v.at[c], sem).wait()

        lanes = jnp.arange(_L, dtype=jnp.int32)
        for c in range(n_chunks):
            for b in range(_CHUNK // _L):
                r = rid_v[c, pl.ds(b * _L, _L)]
                q4 = r * 4
                t3 = r * 4
                pq = (c * 512 + b * 64) + lanes * 4
                for j in range(4):
                    p = pq + j
                    plsc.store_scatter(
                        qidx_v, [p >> 7, p & 127], q4 + j)
                pt = (c * 384 + b * 48) + lanes * 3
                for j in range(3):
                    p = pt + j
                    plsc.store_scatter(
                        tidx_v, [p >> 7, p & 127], t3 + j)

        copies = []
        for rr in range(qrows_w):
            copies.append(
                pltpu.async_copy(qf_hbm.at[qidx_v.at[rr]], qg_v.at[rr], sem))
        for rr in range(trows_w):
            copies.append(
                pltpu.async_copy(tf_hbm.at[tidx_v.at[rr]], tg_v.at[rr], sem))
        for cp in copies:
            cp.wait()
        pltpu.sync_copy(qg_v, out_q.at[pl.ds(wid * qrows_w, qrows_w)])
        pltpu.sync_copy(tg_v, out_t.at[pl.ds(wid * trows_w, trows_w)])

    _cached = cam_gather
    return cam_gather


def kernel(quat, trans, frame_mapping_inv, frame_id):
    qf = lax.optimization_barrier(quat.reshape(31250, _CHUNK)).reshape(-1)
    tf = lax.optimization_barrier(
        jnp.pad(trans, ((0, 0), (0, 1))).reshape(31250, _CHUNK)).reshape(-1)
    fid2d = frame_id.reshape(_M // _CHUNK, _CHUNK)
    out_q, out_t = _build()(qf, tf, frame_mapping_inv, fid2d)
    return out_q.reshape(_M, 4), out_t.reshape(_M, 3)


# speedup vs baseline: 3.2463x; 3.2463x over previous
import functools

import jax
import jax.numpy as jnp
from jax import lax
from jax.experimental import pallas as pl
from jax.experimental.pallas import tpu as pltpu
from jax.experimental.pallas import tpu_sc as plsc

_N = 1000000
_M = 16384
_CHUNK = 128
_L = 16

_cached = None


def _build():
    global _cached
    if _cached is not None:
        return _cached

    info = plsc.get_sparse_core_info()
    NC, NS = info.num_cores, info.num_subcores
    NW = NC * NS
    assert _M % (NW * _CHUNK) == 0
    n_chunks = _M // (NW * _CHUNK)
    ids_w = n_chunks * _CHUNK
    qrows_w = ids_w * 4 // _CHUNK
    trows_w = ids_w * 3 // _CHUNK

    mesh = plsc.VectorSubcoreMesh(core_axis_name="c", subcore_axis_name="s")

    @functools.partial(
        pl.kernel,
        mesh=mesh,
        out_type=(
            jax.ShapeDtypeStruct((_M * 4 // _CHUNK, _CHUNK), jnp.float32),
            jax.ShapeDtypeStruct((_M * 3 // _CHUNK, _CHUNK), jnp.float32),
        ),
        scratch_types=[
            pltpu.VMEM((n_chunks, _CHUNK), jnp.int32),
            pltpu.VMEM((n_chunks, _CHUNK), jnp.int32),
            pltpu.VMEM((qrows_w, _CHUNK), jnp.int32),
            pltpu.VMEM((trows_w, _CHUNK), jnp.int32),
            pltpu.VMEM((qrows_w, _CHUNK), jnp.float32),
            pltpu.VMEM((trows_w, _CHUNK), jnp.float32),
            pltpu.SemaphoreType.DMA,
        ],
        compiler_params=pltpu.CompilerParams(
            use_tc_tiling_on_sc=False, needs_layout_passes=False),
    )
    def cam_gather(qf_hbm, tf_hbm, fmi_hbm, fid_hbm, out_q, out_t,
                   idx_v, rid_v, qidx_v, tidx_v, qg_v, tg_v, sem):
        wid = lax.axis_index("s") * NC + lax.axis_index("c")
        pltpu.sync_copy(fid_hbm.at[pl.ds(wid * n_chunks, n_chunks)], idx_v)
        for c in range(n_chunks):
            pltpu.async_copy(fmi_hbm.at[idx_v.at[c]], rid_v.at[c], sem).wait()

        lanes = jnp.arange(_L, dtype=jnp.int32)
        for c in range(n_chunks):
            for b in range(_CHUNK // _L):
                r = rid_v[c, pl.ds(b * _L, _L)]
                q4 = r * 4
                t3 = r
                pq = (c * 512 + b * 64) + lanes * 4
                for j in range(4):
                    p = pq + j
                    plsc.store_scatter(
                        qidx_v, [p >> 7, p & 127], q4 + j)
                pt = (c * 384 + b * 48) + lanes * 3
                for j in range(3):
                    p = pt + j
                    plsc.store_scatter(
                        tidx_v, [p >> 7, p & 127], t3 + j * _N)

        copies = []
        for rr in range(qrows_w):
            copies.append(
                pltpu.async_copy(qf_hbm.at[qidx_v.at[rr]], qg_v.at[rr], sem))
        for rr in range(trows_w):
            copies.append(
                pltpu.async_copy(tf_hbm.at[tidx_v.at[rr]], tg_v.at[rr], sem))
        for cp in copies:
            cp.wait()
        pltpu.sync_copy(qg_v, out_q.at[pl.ds(wid * qrows_w, qrows_w)])
        pltpu.sync_copy(tg_v, out_t.at[pl.ds(wid * trows_w, trows_w)])

    _cached = cam_gather
    return cam_gather


def kernel(quat, trans, frame_mapping_inv, frame_id):
    qf = lax.optimization_barrier(quat.reshape(31250, _CHUNK)).reshape(-1)
    tf = lax.optimization_barrier(trans.T).reshape(-1)
    fid2d = frame_id.reshape(_M // _CHUNK, _CHUNK)
    out_q, out_t = _build()(qf, tf, frame_mapping_inv, fid2d)
    return out_q.reshape(_M, 4), out_t.reshape(_M, 3)


# speedup vs baseline: 35.3485x; 10.8887x over previous
import functools

import jax
import jax.numpy as jnp
from jax import lax
from jax.experimental import pallas as pl
from jax.experimental.pallas import tpu as pltpu
from jax.experimental.pallas import tpu_sc as plsc

_N = 1000000
_M = 16384
_CHUNK = 128
_L = 16

_cached = None


def _build():
    global _cached
    if _cached is not None:
        return _cached

    info = plsc.get_sparse_core_info()
    NC, NS = info.num_cores, info.num_subcores
    NW = NC * NS
    assert _M % (NW * _CHUNK) == 0
    n_chunks = _M // (NW * _CHUNK)
    ids_w = n_chunks * _CHUNK
    qrows_w = ids_w * 4 // _CHUNK
    trows_w = ids_w * 3 // _CHUNK

    mesh = plsc.VectorSubcoreMesh(core_axis_name="c", subcore_axis_name="s")

    @functools.partial(
        pl.kernel,
        mesh=mesh,
        out_type=(
            jax.ShapeDtypeStruct((_M * 4 // _CHUNK, _CHUNK), jnp.float32),
            jax.ShapeDtypeStruct((_M * 3 // _CHUNK, _CHUNK), jnp.float32),
        ),
        scratch_types=[
            pltpu.VMEM((n_chunks, _CHUNK), jnp.int32),
            pltpu.VMEM((n_chunks, _CHUNK), jnp.int32),
            pltpu.VMEM((qrows_w, _CHUNK), jnp.int32),
            pltpu.VMEM((trows_w, _CHUNK), jnp.int32),
            pltpu.VMEM((qrows_w, _CHUNK), jnp.float32),
            pltpu.VMEM((trows_w, _CHUNK), jnp.float32),
            pltpu.SemaphoreType.DMA,
        ],
        compiler_params=pltpu.CompilerParams(
            use_tc_tiling_on_sc=False, needs_layout_passes=False),
    )
    def cam_gather(qf_hbm, tf_hbm, fmi_hbm, fid_hbm, out_q, out_t,
                   idx_v, rid_v, qidx_v, tidx_v, qg_v, tg_v, sem):
        wid = lax.axis_index("s") * NC + lax.axis_index("c")
        pltpu.sync_copy(fid_hbm.at[pl.ds(wid * n_chunks, n_chunks)], idx_v)
        for c in range(n_chunks):
            pltpu.async_copy(fmi_hbm.at[idx_v.at[c]], rid_v.at[c], sem).wait()

        lanes = jnp.arange(_L, dtype=jnp.int32)
        for c in range(n_chunks):
            for b in range(_CHUNK // _L):
                r = rid_v[c, pl.ds(b * _L, _L)]
                q4 = r
                t3 = r
                pq = (c * 512 + b * 64) + lanes * 4
                for j in range(4):
                    p = pq + j
                    plsc.store_scatter(
                        qidx_v, [p >> 7, p & 127], q4 + j * _N)
                pt = (c * 384 + b * 48) + lanes * 3
                for j in range(3):
                    p = pt + j
                    plsc.store_scatter(
                        tidx_v, [p >> 7, p & 127], t3 + j * _N)

        copies = []
        for rr in range(qrows_w):
            copies.append(
                pltpu.async_copy(qf_hbm.at[qidx_v.at[rr]], qg_v.at[rr], sem))
        for rr in range(trows_w):
            copies.append(
                pltpu.async_copy(tf_hbm.at[tidx_v.at[rr]], tg_v.at[rr], sem))
        for cp in copies:
            cp.wait()
        pltpu.sync_copy(qg_v, out_q.at[pl.ds(wid * qrows_w, qrows_w)])
        pltpu.sync_copy(tg_v, out_t.at[pl.ds(wid * trows_w, trows_w)])

    _cached = cam_gather
    return cam_gather


def kernel(quat, trans, frame_mapping_inv, frame_id):
    qf = lax.optimization_barrier(quat.T).reshape(-1)
    tf = lax.optimization_barrier(trans.T).reshape(-1)
    fid2d = frame_id.reshape(_M // _CHUNK, _CHUNK)
    out_q, out_t = _build()(qf, tf, frame_mapping_inv, fid2d)
    return out_q.reshape(_M, 4), out_t.reshape(_M, 3)


# speedup vs baseline: 35.4861x; 1.0039x over previous
import functools

import jax
import jax.numpy as jnp
from jax import lax
from jax.experimental import pallas as pl
from jax.experimental.pallas import tpu as pltpu
from jax.experimental.pallas import tpu_sc as plsc

_N = 1000000
_M = 16384
_CHUNK = 128
_L = 16

_cached = None


def _build():
    global _cached
    if _cached is not None:
        return _cached

    info = plsc.get_sparse_core_info()
    NC, NS = info.num_cores, info.num_subcores
    NW = NC * NS
    assert _M % (NW * _CHUNK) == 0
    n_chunks = _M // (NW * _CHUNK)
    ids_w = n_chunks * _CHUNK
    qrows_w = ids_w * 4 // _CHUNK
    trows_w = ids_w * 3 // _CHUNK

    mesh = plsc.VectorSubcoreMesh(core_axis_name="c", subcore_axis_name="s")

    @functools.partial(
        pl.kernel,
        mesh=mesh,
        out_type=(
            jax.ShapeDtypeStruct((_M * 4 // _CHUNK, _CHUNK), jnp.float32),
            jax.ShapeDtypeStruct((_M * 3 // _CHUNK, _CHUNK), jnp.float32),
        ),
        scratch_types=[
            pltpu.VMEM((n_chunks, _CHUNK), jnp.int32),
            pltpu.VMEM((n_chunks, _CHUNK), jnp.int32),
            pltpu.VMEM((qrows_w, _CHUNK), jnp.int32),
            pltpu.VMEM((trows_w, _CHUNK), jnp.int32),
            pltpu.VMEM((qrows_w, _CHUNK), jnp.float32),
            pltpu.VMEM((trows_w, _CHUNK), jnp.float32),
            pltpu.SemaphoreType.DMA,
        ],
        compiler_params=pltpu.CompilerParams(
            use_tc_tiling_on_sc=False, needs_layout_passes=False),
    )
    def cam_gather(qf_hbm, tf_hbm, fmi_hbm, fid_hbm, out_q, out_t,
                   idx_v, rid_v, qidx_v, tidx_v, qg_v, tg_v, sem):
        wid = lax.axis_index("s") * NC + lax.axis_index("c")
        pltpu.sync_copy(fid_hbm.at[pl.ds(wid * n_chunks, n_chunks)], idx_v)
        for c in range(n_chunks):
            pltpu.async_copy(fmi_hbm.at[idx_v.at[c]], rid_v.at[c], sem).wait()

        lanes = jnp.arange(_L, dtype=jnp.int32)
        for c in range(n_chunks):
            for b in range(_CHUNK // _L):
                r = rid_v[c, pl.ds(b * _L, _L)]
                q4 = r
                t3 = r
                pq = (c * 512 + b * 64) + lanes * 4
                for j in range(4):
                    p = pq + j
                    plsc.store_scatter(
                        qidx_v, [p >> 7, p & 127], q4 + j * _N)
                pt = (c * 384 + b * 48) + lanes * 3
                for j in range(3):
                    p = pt + j
                    plsc.store_scatter(
                        tidx_v, [p >> 7, p & 127], t3 + j * _N)

        copies = []
        for rr in range(qrows_w):
            copies.append(
                pltpu.async_copy(qf_hbm.at[qidx_v.at[rr]], qg_v.at[rr], sem))
        for rr in range(trows_w):
            copies.append(
                pltpu.async_copy(tf_hbm.at[tidx_v.at[rr]], tg_v.at[rr], sem))
        for cp in copies:
            cp.wait()
        pltpu.sync_copy(qg_v, out_q.at[pl.ds(wid * qrows_w, qrows_w)])
        pltpu.sync_copy(tg_v, out_t.at[pl.ds(wid * trows_w, trows_w)])

    _cached = cam_gather
    return cam_gather


def kernel(quat, trans, frame_mapping_inv, frame_id):
    qf = quat.T.reshape(-1)
    tf = trans.T.reshape(-1)
    fid2d = frame_id.reshape(_M // _CHUNK, _CHUNK)
    out_q, out_t = _build()(qf, tf, frame_mapping_inv, fid2d)
    return out_q.reshape(_M, 4), out_t.reshape(_M, 3)
